# Initial kernel scaffold; baseline (speedup 1.0000x reference)
import jax
import jax.numpy as jnp
from jax.experimental import pallas as pl


def kernel(x, edge_index, batch, W1, b1, g1, be1, W2, b2, g2, be2, W3, b3, g3, be3, L0, Lb0, L1, Lb1, L2, Lb2, Lo, Lbo):
    raise NotImplementedError("write your pallas kernel here")


# speedup vs baseline: 7.7337x; 7.7337x over previous
import functools

import jax
import jax.numpy as jnp
from jax import lax
from jax.experimental import pallas as pl
from jax.experimental.pallas import tpu as pltpu
from jax.experimental.pallas import tpu_sc as plsc

N = 10000
E = 320000
F_IN = 128
H = 256
LATENT = 128
DENSE = 512
G = 64
EPS = 1e-5

NC = 2
NS = 16
CH = 128
IG = 8
DW = 128
EP = 327680
NROW = 10112
STRIPE = NROW // NS
R = 1000
NBLK = N // R

def _sc_mesh():
    return plsc.VectorSubcoreMesh(core_axis_name="c", subcore_axis_name="s",
                                  num_cores=NC, num_subcores=NS)


def _sc_deg(dst2d, ones_h, zero_h, dw=DW):
    chunks = EP // (NC * NS * CH)
    groups = chunks // IG

    @functools.partial(
        pl.kernel,
        out_type=jax.ShapeDtypeStruct((NC, NROW, dw), jnp.float32),
        mesh=_sc_mesh(),
        scratch_types=[
            pltpu.VMEM((IG, CH), jnp.int32),
            pltpu.VMEM((CH, dw), jnp.float32),
            pltpu.VMEM_SHARED((NROW, dw), jnp.float32),
        ],
    )
    def k(dst_h, ones_hh, zero_hh, out_h, didx, ones_v, acc):
        cid = lax.axis_index("c")
        sid = lax.axis_index("s")
        tid = cid * NS + sid
        pltpu.sync_copy(ones_hh, ones_v)
        pltpu.sync_copy(zero_hh, acc.at[pl.ds(sid * STRIPE, STRIPE)])
        plsc.subcore_barrier()

        def group(gi, carry):
            pltpu.sync_copy(dst_h.at[pl.ds(tid * chunks + gi * IG, IG)], didx)

            def body(j, c2):
                pltpu.sync_copy(ones_v, acc.at[didx.at[j]], add=True)
                return c2

            lax.fori_loop(0, IG, body, 0)
            return carry

        lax.fori_loop(0, groups, group, 0)
        plsc.subcore_barrier()
        pltpu.sync_copy(acc.at[pl.ds(sid * STRIPE, STRIPE)],
                        out_h.at[cid, pl.ds(sid * STRIPE, STRIPE)])

    return k(dst2d, ones_h, zero_h)


def _make_agg(feat_split):
    chunks = EP // (NS * CH) if feat_split else EP // (NC * NS * CH)
    groups = chunks // IG

    @functools.partial(
        pl.kernel,
        out_type=jax.ShapeDtypeStruct((NC, NROW, 128), jnp.float32),
        mesh=_sc_mesh(),
        scratch_types=[
            pltpu.VMEM((IG, CH), jnp.int32),
            pltpu.VMEM((IG, CH), jnp.int32),
            pltpu.VMEM((CH, 128), jnp.float32),
            pltpu.VMEM_SHARED((NROW, 128), jnp.float32),
            pltpu.SemaphoreType.DMA,
        ],
    )
    def k(table_h, src_h, dst_h, zero_hh, out_h, sidx, didx, rows, acc, sem):
        cid = lax.axis_index("c")
        sid = lax.axis_index("s")
        if feat_split:
            src_base = cid * (EP // CH) + sid * chunks
            dst_base = sid * chunks
        else:
            tid = cid * NS + sid
            src_base = tid * chunks
            dst_base = tid * chunks
        pltpu.sync_copy(zero_hh, acc.at[pl.ds(sid * STRIPE, STRIPE)])
        plsc.subcore_barrier()

        def group(gi, carry):
            pltpu.sync_copy(src_h.at[pl.ds(src_base + gi * IG, IG)], sidx)
            pltpu.sync_copy(dst_h.at[pl.ds(dst_base + gi * IG, IG)], didx)

            def body(j, c2):
                pltpu.async_copy(table_h.at[sidx.at[j]], rows, sem).wait()
                pltpu.sync_copy(rows, acc.at[didx.at[j]], add=True)
                return c2

            lax.fori_loop(0, IG, body, 0)
            return carry

        lax.fori_loop(0, groups, group, 0)
        plsc.subcore_barrier()
        pltpu.sync_copy(acc.at[pl.ds(sid * STRIPE, STRIPE)],
                        out_h.at[cid, pl.ds(sid * STRIPE, STRIPE)])

    return k


def _tc_prep1(x, degp):
    def body(x_ref, d_ref, dinv_ref, u_ref):
        d = d_ref[...]
        dinv = lax.rsqrt(d[0, :, 0:1] + d[1, :, 0:1] + 1.0)
        dinv_ref[...] = dinv
        u_ref[...] = x_ref[...] * dinv

    return pl.pallas_call(
        body,
        grid=(NBLK,),
        in_specs=[
            pl.BlockSpec((R, F_IN), lambda i: (i, 0)),
            pl.BlockSpec((NC, R, DW), lambda i: (0, i, 0)),
        ],
        out_specs=[
            pl.BlockSpec((R, 1), lambda i: (i, 0)),
            pl.BlockSpec((R, F_IN), lambda i: (i, 0)),
        ],
        out_shape=[
            jax.ShapeDtypeStruct((N, 1), jnp.float32),
            jax.ShapeDtypeStruct((N, F_IN), jnp.float32),
        ],
    )(x, degp)


def _tc_layer_mm(parts, u, dinv, W, b, split_u, fout):
    fin = 256 if split_u else 128

    def body(p_ref, u_ref, dinv_ref, W_ref, b_ref, z_ref, st_ref, acc_ref):
        i = pl.program_id(0)
        p = p_ref[...]
        dinv = dinv_ref[...]
        if split_u:
            uu = u_ref[...]
            y = jnp.concatenate([(p[0] + uu[0]) * dinv,
                                 (p[1] + uu[1]) * dinv], axis=1)
        else:
            y = (p[0] + p[1] + u_ref[...]) * dinv
        z = jnp.dot(y, W_ref[...], preferred_element_type=jnp.float32, precision=lax.Precision.HIGHEST) + b_ref[...]
        z_ref[...] = z

        @pl.when(i == 0)
        def _():
            acc_ref[...] = jnp.zeros((8, fout), jnp.float32)

        acc_ref[0:1] += jnp.sum(z, axis=0, keepdims=True)
        acc_ref[1:2] += jnp.sum(z * z, axis=0, keepdims=True)

        @pl.when(i == NBLK - 1)
        def _():
            st_ref[...] = acc_ref[...]

    u_spec = (pl.BlockSpec((NC, R, 128), lambda i: (0, i, 0)) if split_u
              else pl.BlockSpec((R, 128), lambda i: (i, 0)))
    return pl.pallas_call(
        body,
        grid=(NBLK,),
        in_specs=[
            pl.BlockSpec((NC, R, 128), lambda i: (0, i, 0)),
            u_spec,
            pl.BlockSpec((R, 1), lambda i: (i, 0)),
            pl.BlockSpec((fin, fout), lambda i: (0, 0)),
            pl.BlockSpec((1, fout), lambda i: (0, 0)),
        ],
        out_specs=[
            pl.BlockSpec((R, fout), lambda i: (i, 0)),
            pl.BlockSpec((8, fout), lambda i: (0, 0)),
        ],
        out_shape=[
            jax.ShapeDtypeStruct((N, fout), jnp.float32),
            jax.ShapeDtypeStruct((8, fout), jnp.float32),
        ],
        scratch_shapes=[pltpu.VMEM((8, fout), jnp.float32)],
    )(parts, u, dinv, W, b)


def _tc_bn_prep2(z, st, g, be, dinv):
    def body(z_ref, st_ref, g_ref, be_ref, dinv_ref, u_ref):
        st = st_ref[...]
        mu = st[0:1] / N
        var = st[1:2] / N - mu * mu
        alpha = g_ref[...] * lax.rsqrt(var + EPS)
        beta = be_ref[...] - mu * alpha
        h = jnp.maximum(z_ref[...] * alpha + beta, 0.0)
        u = h * dinv_ref[...]
        u_ref[...] = jnp.stack([u[:, :128], u[:, 128:]], axis=0)

    return pl.pallas_call(
        body,
        grid=(NBLK,),
        in_specs=[
            pl.BlockSpec((R, H), lambda i: (i, 0)),
            pl.BlockSpec((8, H), lambda i: (0, 0)),
            pl.BlockSpec((1, H), lambda i: (0, 0)),
            pl.BlockSpec((1, H), lambda i: (0, 0)),
            pl.BlockSpec((R, 1), lambda i: (i, 0)),
        ],
        out_specs=pl.BlockSpec((NC, R, 128), lambda i: (0, i, 0)),
        out_shape=jax.ShapeDtypeStruct((NC, N, 128), jnp.float32),
    )(z, st, g, be, dinv)


def _tc_bn_mm_prep3(z, st, g, be, dinv, W3):
    def body(z_ref, st_ref, g_ref, be_ref, dinv_ref, W_ref, u_ref):
        st = st_ref[...]
        mu = st[0:1] / N
        var = st[1:2] / N - mu * mu
        alpha = g_ref[...] * lax.rsqrt(var + EPS)
        beta = be_ref[...] - mu * alpha
        h = jnp.maximum(z_ref[...] * alpha + beta, 0.0)
        t = jnp.dot(h, W_ref[...], preferred_element_type=jnp.float32, precision=lax.Precision.HIGHEST)
        u_ref[...] = t * dinv_ref[...]

    return pl.pallas_call(
        body,
        grid=(NBLK,),
        in_specs=[
            pl.BlockSpec((R, H), lambda i: (i, 0)),
            pl.BlockSpec((8, H), lambda i: (0, 0)),
            pl.BlockSpec((1, H), lambda i: (0, 0)),
            pl.BlockSpec((1, H), lambda i: (0, 0)),
            pl.BlockSpec((R, 1), lambda i: (i, 0)),
            pl.BlockSpec((H, LATENT), lambda i: (0, 0)),
        ],
        out_specs=pl.BlockSpec((R, LATENT), lambda i: (i, 0)),
        out_shape=jax.ShapeDtypeStruct((N, LATENT), jnp.float32),
    )(z, st, g, be, dinv, W3)


def _tc_layer3_pool(parts, u3, dinv, b3, batch2d):
    def body(p_ref, u_ref, dinv_ref, b_ref, bat_ref, gs_ref, cnt_ref, st_ref,
             gacc, cacc, sacc):
        i = pl.program_id(0)

        @pl.when(i == 0)
        def _():
            gacc[...] = jnp.zeros((G, LATENT), jnp.float32)
            cacc[...] = jnp.zeros((G, 1), jnp.float32)
            sacc[...] = jnp.zeros((8, LATENT), jnp.float32)

        p = p_ref[...]
        y = (p[0] + p[1] + u_ref[...]) * dinv_ref[...] + b_ref[...]
        sacc[0:1] += jnp.sum(y, axis=0, keepdims=True)
        sacc[1:2] += jnp.sum(y * y, axis=0, keepdims=True)
        bat = bat_ref[0]
        oh = (lax.broadcasted_iota(jnp.int32, (G, R), 0)
              == jnp.broadcast_to(bat, (G, R))).astype(jnp.float32)
        gacc[...] += jnp.dot(oh, y, preferred_element_type=jnp.float32, precision=lax.Precision.HIGHEST)
        cacc[...] += jnp.sum(oh, axis=1, keepdims=True)

        @pl.when(i == NBLK - 1)
        def _():
            gs_ref[...] = gacc[...]
            cnt_ref[...] = cacc[...]
            st_ref[...] = sacc[...]

    return pl.pallas_call(
        body,
        grid=(NBLK,),
        in_specs=[
            pl.BlockSpec((NC, R, 128), lambda i: (0, i, 0)),
            pl.BlockSpec((R, LATENT), lambda i: (i, 0)),
            pl.BlockSpec((R, 1), lambda i: (i, 0)),
            pl.BlockSpec((1, LATENT), lambda i: (0, 0)),
            pl.BlockSpec((1, 1, R), lambda i: (i, 0, 0)),
        ],
        out_specs=[
            pl.BlockSpec((G, LATENT), lambda i: (0, 0)),
            pl.BlockSpec((G, 1), lambda i: (0, 0)),
            pl.BlockSpec((8, LATENT), lambda i: (0, 0)),
        ],
        out_shape=[
            jax.ShapeDtypeStruct((G, LATENT), jnp.float32),
            jax.ShapeDtypeStruct((G, 1), jnp.float32),
            jax.ShapeDtypeStruct((8, LATENT), jnp.float32),
        ],
        scratch_shapes=[
            pltpu.VMEM((G, LATENT), jnp.float32),
            pltpu.VMEM((G, 1), jnp.float32),
            pltpu.VMEM((8, LATENT), jnp.float32),
        ],
    )(parts, u3, dinv, b3, batch2d)


def _tc_head(gsum, cnt, st, g3, be3, L0, Lb0, L1, Lb1, L2, Lb2, Lo_p, Lbo_p):
    def body(gs_ref, cnt_ref, st_ref, g_ref, be_ref, L0_ref, Lb0_ref, L1_ref,
             Lb1_ref, L2_ref, Lb2_ref, Lo_ref, Lbo_ref, o_ref):
        st = st_ref[...]
        mu = st[0:1] / N
        var = st[1:2] / N - mu * mu
        alpha = g_ref[...] * lax.rsqrt(var + EPS)
        beta = be_ref[...] - mu * alpha
        pooled = gs_ref[...] / jnp.maximum(cnt_ref[...], 1.0)
        hb = pooled * alpha + beta
        z = jnp.maximum(jnp.dot(hb, L0_ref[...], preferred_element_type=jnp.float32, precision=lax.Precision.HIGHEST) + Lb0_ref[...], 0.0)
        z = jnp.maximum(jnp.dot(z, L1_ref[...], preferred_element_type=jnp.float32, precision=lax.Precision.HIGHEST) + Lb1_ref[...], 0.0)
        z = jnp.maximum(jnp.dot(z, L2_ref[...], preferred_element_type=jnp.float32, precision=lax.Precision.HIGHEST) + Lb2_ref[...], 0.0)
        o_ref[...] = jnp.dot(z, Lo_ref[...], preferred_element_type=jnp.float32, precision=lax.Precision.HIGHEST) + Lbo_ref[...]

    return pl.pallas_call(
        body,
        out_shape=jax.ShapeDtypeStruct((G, 128), jnp.float32),
    )(gsum, cnt, st, g3, be3, L0, Lb0, L1, Lb1, L2, Lb2, Lo_p, Lbo_p)


def kernel(x, edge_index, batch, W1, b1, g1, be1, W2, b2, g2, be2, W3, b3,
           g3, be3, L0, Lb0, L1, Lb1, L2, Lb2, Lo, Lbo):
    src = edge_index[0]
    dst = edge_index[1]
    pad = EP - E
    src_p = jnp.concatenate([src, jnp.zeros((pad,), jnp.int32)])
    dst_p = jnp.concatenate([dst, jnp.full((pad,), N, jnp.int32)])
    src2d = src_p.reshape(EP // CH, CH)
    dst2d = dst_p.reshape(EP // CH, CH)
    src_off2d = jnp.concatenate([src_p, src_p + N]).reshape(2 * EP // CH, CH)

    ones_c = jnp.ones((CH, DW), jnp.float32)
    zero1 = jnp.zeros((STRIPE, DW), jnp.float32)
    zero128 = jnp.zeros((STRIPE, 128), jnp.float32)
    batch3d = batch.reshape(NBLK, 1, R)

    b1r, b2r, b3r = b1.reshape(1, H), b2.reshape(1, H), b3.reshape(1, LATENT)
    g1r, g2r, g3r = g1.reshape(1, H), g2.reshape(1, H), g3.reshape(1, LATENT)
    be1r, be2r, be3r = be1.reshape(1, H), be2.reshape(1, H), be3.reshape(1, LATENT)
    Lb0r, Lb1r, Lb2r = Lb0.reshape(1, DENSE), Lb1.reshape(1, DENSE), Lb2.reshape(1, DENSE)
    Lo_p = jnp.pad(Lo, ((0, 0), (0, 128 - Lo.shape[1])))
    Lbo_p = jnp.pad(Lbo, (0, 128 - Lbo.shape[0])).reshape(1, 128)

    agg_es = _make_agg(feat_split=False)
    agg_fs = _make_agg(feat_split=True)

    degp = _sc_deg(dst2d, ones_c, zero1)
    dinv, u1 = _tc_prep1(x, degp)

    p1 = agg_es(u1, src2d, dst2d, zero128)
    z1, st1 = _tc_layer_mm(p1, u1, dinv, W1, b1r, split_u=False, fout=H)

    u2 = _tc_bn_prep2(z1, st1, g1r, be1r, dinv)
    p2 = agg_fs(u2.reshape(2 * N, 128), src_off2d, dst2d, zero128)
    z2, st2 = _tc_layer_mm(p2, u2, dinv, W2, b2r, split_u=True, fout=H)

    u3 = _tc_bn_mm_prep3(z2, st2, g2r, be2r, dinv, W3)
    p3 = agg_es(u3, src2d, dst2d, zero128)
    gsum, cnt, st3 = _tc_layer3_pool(p3, u3, dinv, b3r, batch3d)

    out = _tc_head(gsum, cnt, st3, g3r, be3r, L0, Lb0r, L1, Lb1r, L2, Lb2r,
                   Lo_p, Lbo_p)
    return out[:, :3]


# speedup vs baseline: 7.7369x; 1.0004x over previous
import functools

import jax
import jax.numpy as jnp
from jax import lax
from jax.experimental import pallas as pl
from jax.experimental.pallas import tpu as pltpu
from jax.experimental.pallas import tpu_sc as plsc

N = 10000
E = 320000
F_IN = 128
H = 256
LATENT = 128
DENSE = 512
G = 64
EPS = 1e-5

NC = 2
NS = 16
CH = 128
IG = 8
DW = 128
EP = 327680
NROW = 10112
STRIPE = NROW // NS
R = 1000
NBLK = N // R

def _sc_mesh():
    return plsc.VectorSubcoreMesh(core_axis_name="c", subcore_axis_name="s",
                                  num_cores=NC, num_subcores=NS)


def _sc_deg(dst2d, ones_h, zero_h, dw=DW):
    chunks = EP // (NC * NS * CH)
    groups = chunks // IG

    @functools.partial(
        pl.kernel,
        out_type=jax.ShapeDtypeStruct((NC, NROW, dw), jnp.float32),
        mesh=_sc_mesh(),
        scratch_types=[
            pltpu.VMEM((IG, CH), jnp.int32),
            pltpu.VMEM((CH, dw), jnp.float32),
            pltpu.VMEM_SHARED((NROW, dw), jnp.float32),
        ],
    )
    def k(dst_h, ones_hh, zero_hh, out_h, didx, ones_v, acc):
        cid = lax.axis_index("c")
        sid = lax.axis_index("s")
        tid = cid * NS + sid
        pltpu.sync_copy(ones_hh, ones_v)
        pltpu.sync_copy(zero_hh, acc.at[pl.ds(sid * STRIPE, STRIPE)])
        plsc.subcore_barrier()

        def group(gi, carry):
            pltpu.sync_copy(dst_h.at[pl.ds(tid * chunks + gi * IG, IG)], didx)

            def body(j, c2):
                pltpu.sync_copy(ones_v, acc.at[didx.at[j]], add=True)
                return c2

            lax.fori_loop(0, IG, body, 0)
            return carry

        lax.fori_loop(0, groups, group, 0)
        plsc.subcore_barrier()
        pltpu.sync_copy(acc.at[pl.ds(sid * STRIPE, STRIPE)],
                        out_h.at[cid, pl.ds(sid * STRIPE, STRIPE)])

    return k(dst2d, ones_h, zero_h)


def _make_agg(feat_split):
    chunks = EP // (NS * CH) if feat_split else EP // (NC * NS * CH)
    groups = chunks // IG

    @functools.partial(
        pl.kernel,
        out_type=jax.ShapeDtypeStruct((NC, NROW, 128), jnp.float32),
        mesh=_sc_mesh(),
        scratch_types=[
            pltpu.VMEM((IG, CH), jnp.int32),
            pltpu.VMEM((IG, CH), jnp.int32),
            pltpu.VMEM((CH, 128), jnp.float32),
            pltpu.VMEM_SHARED((NROW, 128), jnp.float32),
            pltpu.SemaphoreType.DMA,
        ],
    )
    def k(table_h, src_h, dst_h, zero_hh, out_h, sidx, didx, rows, acc, sem):
        cid = lax.axis_index("c")
        sid = lax.axis_index("s")
        if feat_split:
            src_base = cid * (EP // CH) + sid * chunks
            dst_base = sid * chunks
        else:
            tid = cid * NS + sid
            src_base = tid * chunks
            dst_base = tid * chunks
        pltpu.sync_copy(zero_hh, acc.at[pl.ds(sid * STRIPE, STRIPE)])
        plsc.subcore_barrier()

        def group(gi, carry):
            pltpu.sync_copy(src_h.at[pl.ds(src_base + gi * IG, IG)], sidx)
            pltpu.sync_copy(dst_h.at[pl.ds(dst_base + gi * IG, IG)], didx)

            def body(j, c2):
                pltpu.async_copy(table_h.at[sidx.at[j]], rows, sem).wait()
                pltpu.sync_copy(rows, acc.at[didx.at[j]], add=True)
                return c2

            lax.fori_loop(0, IG, body, 0)
            return carry

        lax.fori_loop(0, groups, group, 0)
        plsc.subcore_barrier()
        pltpu.sync_copy(acc.at[pl.ds(sid * STRIPE, STRIPE)],
                        out_h.at[cid, pl.ds(sid * STRIPE, STRIPE)])

    return k


def _tc_prep1(x, degp):
    def body(x_ref, d_ref, dinv_ref, u_ref):
        d = d_ref[...]
        dinv = lax.rsqrt(d[0, :, 0:1] + d[1, :, 0:1] + 1.0)
        dinv_ref[...] = dinv
        u_ref[...] = x_ref[...] * dinv

    return pl.pallas_call(
        body,
        grid=(NBLK,),
        in_specs=[
            pl.BlockSpec((R, F_IN), lambda i: (i, 0)),
            pl.BlockSpec((NC, R, DW), lambda i: (0, i, 0)),
        ],
        out_specs=[
            pl.BlockSpec((R, 1), lambda i: (i, 0)),
            pl.BlockSpec((R, F_IN), lambda i: (i, 0)),
        ],
        out_shape=[
            jax.ShapeDtypeStruct((N, 1), jnp.float32),
            jax.ShapeDtypeStruct((N, F_IN), jnp.float32),
        ],
    )(x, degp)


def _tc_layer_mm(parts, u, dinv, W, b, split_u, fout):
    fin = 256 if split_u else 128

    def body(p_ref, u_ref, dinv_ref, W_ref, b_ref, z_ref, st_ref, acc_ref):
        i = pl.program_id(0)
        p = p_ref[...]
        dinv = dinv_ref[...]
        if split_u:
            uu = u_ref[...]
            y = jnp.concatenate([(p[0] + uu[0]) * dinv,
                                 (p[1] + uu[1]) * dinv], axis=1)
        else:
            y = (p[0] + p[1] + u_ref[...]) * dinv
        z = jnp.dot(y, W_ref[...], preferred_element_type=jnp.float32, precision=lax.Precision.HIGHEST) + b_ref[...]
        z_ref[...] = z

        @pl.when(i == 0)
        def _():
            acc_ref[...] = jnp.zeros((8, fout), jnp.float32)

        acc_ref[0:1] += jnp.sum(z, axis=0, keepdims=True)
        acc_ref[1:2] += jnp.sum(z * z, axis=0, keepdims=True)

        @pl.when(i == NBLK - 1)
        def _():
            st_ref[...] = acc_ref[...]

    u_spec = (pl.BlockSpec((NC, R, 128), lambda i: (0, i, 0)) if split_u
              else pl.BlockSpec((R, 128), lambda i: (i, 0)))
    return pl.pallas_call(
        body,
        grid=(NBLK,),
        in_specs=[
            pl.BlockSpec((NC, R, 128), lambda i: (0, i, 0)),
            u_spec,
            pl.BlockSpec((R, 1), lambda i: (i, 0)),
            pl.BlockSpec((fin, fout), lambda i: (0, 0)),
            pl.BlockSpec((1, fout), lambda i: (0, 0)),
        ],
        out_specs=[
            pl.BlockSpec((R, fout), lambda i: (i, 0)),
            pl.BlockSpec((8, fout), lambda i: (0, 0)),
        ],
        out_shape=[
            jax.ShapeDtypeStruct((N, fout), jnp.float32),
            jax.ShapeDtypeStruct((8, fout), jnp.float32),
        ],
        scratch_shapes=[pltpu.VMEM((8, fout), jnp.float32)],
    )(parts, u, dinv, W, b)


def _tc_bn_prep2(z, st, g, be, dinv):
    def body(z_ref, st_ref, g_ref, be_ref, dinv_ref, u_ref):
        st = st_ref[...]
        mu = st[0:1] / N
        var = st[1:2] / N - mu * mu
        alpha = g_ref[...] * lax.rsqrt(var + EPS)
        beta = be_ref[...] - mu * alpha
        h = jnp.maximum(z_ref[...] * alpha + beta, 0.0)
        u = h * dinv_ref[...]
        u_ref[...] = jnp.stack([u[:, :128], u[:, 128:]], axis=0)

    return pl.pallas_call(
        body,
        grid=(NBLK,),
        in_specs=[
            pl.BlockSpec((R, H), lambda i: (i, 0)),
            pl.BlockSpec((8, H), lambda i: (0, 0)),
            pl.BlockSpec((1, H), lambda i: (0, 0)),
            pl.BlockSpec((1, H), lambda i: (0, 0)),
            pl.BlockSpec((R, 1), lambda i: (i, 0)),
        ],
        out_specs=pl.BlockSpec((NC, R, 128), lambda i: (0, i, 0)),
        out_shape=jax.ShapeDtypeStruct((NC, N, 128), jnp.float32),
    )(z, st, g, be, dinv)


def _tc_bn_mm_prep3(z, st, g, be, dinv, W3):
    def body(z_ref, st_ref, g_ref, be_ref, dinv_ref, W_ref, u_ref):
        st = st_ref[...]
        mu = st[0:1] / N
        var = st[1:2] / N - mu * mu
        alpha = g_ref[...] * lax.rsqrt(var + EPS)
        beta = be_ref[...] - mu * alpha
        h = jnp.maximum(z_ref[...] * alpha + beta, 0.0)
        t = jnp.dot(h, W_ref[...], preferred_element_type=jnp.float32, precision=lax.Precision.HIGHEST)
        u_ref[...] = t * dinv_ref[...]

    return pl.pallas_call(
        body,
        grid=(NBLK,),
        in_specs=[
            pl.BlockSpec((R, H), lambda i: (i, 0)),
            pl.BlockSpec((8, H), lambda i: (0, 0)),
            pl.BlockSpec((1, H), lambda i: (0, 0)),
            pl.BlockSpec((1, H), lambda i: (0, 0)),
            pl.BlockSpec((R, 1), lambda i: (i, 0)),
            pl.BlockSpec((H, LATENT), lambda i: (0, 0)),
        ],
        out_specs=pl.BlockSpec((R, LATENT), lambda i: (i, 0)),
        out_shape=jax.ShapeDtypeStruct((N, LATENT), jnp.float32),
    )(z, st, g, be, dinv, W3)


def _tc_layer3_pool(parts, u3, dinv, b3, batch2d):
    def body(p_ref, u_ref, dinv_ref, b_ref, bat_ref, gs_ref, cnt_ref, st_ref,
             gacc, cacc, sacc):
        i = pl.program_id(0)

        @pl.when(i == 0)
        def _():
            gacc[...] = jnp.zeros((G, LATENT), jnp.float32)
            cacc[...] = jnp.zeros((G, 1), jnp.float32)
            sacc[...] = jnp.zeros((8, LATENT), jnp.float32)

        p = p_ref[...]
        y = (p[0] + p[1] + u_ref[...]) * dinv_ref[...] + b_ref[...]
        sacc[0:1] += jnp.sum(y, axis=0, keepdims=True)
        sacc[1:2] += jnp.sum(y * y, axis=0, keepdims=True)
        bat = bat_ref[0]
        oh = (lax.broadcasted_iota(jnp.int32, (G, R), 0)
              == jnp.broadcast_to(bat, (G, R))).astype(jnp.float32)
        gacc[...] += jnp.dot(oh, y, preferred_element_type=jnp.float32, precision=lax.Precision.HIGHEST)
        cacc[...] += jnp.sum(oh, axis=1, keepdims=True)

        @pl.when(i == NBLK - 1)
        def _():
            gs_ref[...] = gacc[...]
            cnt_ref[...] = cacc[...]
            st_ref[...] = sacc[...]

    return pl.pallas_call(
        body,
        grid=(NBLK,),
        in_specs=[
            pl.BlockSpec((NC, R, 128), lambda i: (0, i, 0)),
            pl.BlockSpec((R, LATENT), lambda i: (i, 0)),
            pl.BlockSpec((R, 1), lambda i: (i, 0)),
            pl.BlockSpec((1, LATENT), lambda i: (0, 0)),
            pl.BlockSpec((1, 1, R), lambda i: (i, 0, 0)),
        ],
        out_specs=[
            pl.BlockSpec((G, LATENT), lambda i: (0, 0)),
            pl.BlockSpec((G, 1), lambda i: (0, 0)),
            pl.BlockSpec((8, LATENT), lambda i: (0, 0)),
        ],
        out_shape=[
            jax.ShapeDtypeStruct((G, LATENT), jnp.float32),
            jax.ShapeDtypeStruct((G, 1), jnp.float32),
            jax.ShapeDtypeStruct((8, LATENT), jnp.float32),
        ],
        scratch_shapes=[
            pltpu.VMEM((G, LATENT), jnp.float32),
            pltpu.VMEM((G, 1), jnp.float32),
            pltpu.VMEM((8, LATENT), jnp.float32),
        ],
    )(parts, u3, dinv, b3, batch2d)


def _tc_head(gsum, cnt, st, g3, be3, L0, Lb0, L1, Lb1, L2, Lb2, Lo_p, Lbo_p):
    def body(gs_ref, cnt_ref, st_ref, g_ref, be_ref, L0_ref, Lb0_ref, L1_ref,
             Lb1_ref, L2_ref, Lb2_ref, Lo_ref, Lbo_ref, o_ref):
        st = st_ref[...]
        mu = st[0:1] / N
        var = st[1:2] / N - mu * mu
        alpha = g_ref[...] * lax.rsqrt(var + EPS)
        beta = be_ref[...] - mu * alpha
        pooled = gs_ref[...] / jnp.maximum(cnt_ref[...], 1.0)
        hb = pooled * alpha + beta
        z = jnp.maximum(jnp.dot(hb, L0_ref[...], preferred_element_type=jnp.float32, precision=lax.Precision.HIGHEST) + Lb0_ref[...], 0.0)
        z = jnp.maximum(jnp.dot(z, L1_ref[...], preferred_element_type=jnp.float32, precision=lax.Precision.HIGHEST) + Lb1_ref[...], 0.0)
        z = jnp.maximum(jnp.dot(z, L2_ref[...], preferred_element_type=jnp.float32, precision=lax.Precision.HIGHEST) + Lb2_ref[...], 0.0)
        o_ref[...] = jnp.dot(z, Lo_ref[...], preferred_element_type=jnp.float32, precision=lax.Precision.HIGHEST) + Lbo_ref[...]

    return pl.pallas_call(
        body,
        out_shape=jax.ShapeDtypeStruct((G, 128), jnp.float32),
    )(gsum, cnt, st, g3, be3, L0, Lb0, L1, Lb1, L2, Lb2, Lo_p, Lbo_p)


def kernel(x, edge_index, batch, W1, b1, g1, be1, W2, b2, g2, be2, W3, b3,
           g3, be3, L0, Lb0, L1, Lb1, L2, Lb2, Lo, Lbo):
    src = edge_index[0]
    dst = edge_index[1]
    pad = EP - E
    src_p = jnp.concatenate([src, jnp.zeros((pad,), jnp.int32)])
    pad_dst = N + (jnp.arange(pad, dtype=jnp.int32) % (NROW - N))
    dst_p = jnp.concatenate([dst, pad_dst])
    src2d = src_p.reshape(EP // CH, CH)
    dst2d = dst_p.reshape(EP // CH, CH)
    src_off2d = jnp.concatenate([src_p, src_p + N]).reshape(2 * EP // CH, CH)

    ones_c = jnp.ones((CH, DW), jnp.float32)
    zero1 = jnp.zeros((STRIPE, DW), jnp.float32)
    zero128 = jnp.zeros((STRIPE, 128), jnp.float32)
    batch3d = batch.reshape(NBLK, 1, R)

    b1r, b2r, b3r = b1.reshape(1, H), b2.reshape(1, H), b3.reshape(1, LATENT)
    g1r, g2r, g3r = g1.reshape(1, H), g2.reshape(1, H), g3.reshape(1, LATENT)
    be1r, be2r, be3r = be1.reshape(1, H), be2.reshape(1, H), be3.reshape(1, LATENT)
    Lb0r, Lb1r, Lb2r = Lb0.reshape(1, DENSE), Lb1.reshape(1, DENSE), Lb2.reshape(1, DENSE)
    Lo_p = jnp.pad(Lo, ((0, 0), (0, 128 - Lo.shape[1])))
    Lbo_p = jnp.pad(Lbo, (0, 128 - Lbo.shape[0])).reshape(1, 128)

    agg_es = _make_agg(feat_split=False)
    agg_fs = _make_agg(feat_split=True)

    degp = _sc_deg(dst2d, ones_c, zero1)
    dinv, u1 = _tc_prep1(x, degp)

    p1 = agg_es(u1, src2d, dst2d, zero128)
    z1, st1 = _tc_layer_mm(p1, u1, dinv, W1, b1r, split_u=False, fout=H)

    u2 = _tc_bn_prep2(z1, st1, g1r, be1r, dinv)
    p2 = agg_fs(u2.reshape(2 * N, 128), src_off2d, dst2d, zero128)
    z2, st2 = _tc_layer_mm(p2, u2, dinv, W2, b2r, split_u=True, fout=H)

    u3 = _tc_bn_mm_prep3(z2, st2, g2r, be2r, dinv, W3)
    p3 = agg_es(u3, src2d, dst2d, zero128)
    gsum, cnt, st3 = _tc_layer3_pool(p3, u3, dinv, b3r, batch3d)

    out = _tc_head(gsum, cnt, st3, g3r, be3r, L0, Lb0r, L1, Lb1r, L2, Lb2r,
                   Lo_p, Lbo_p)
    return out[:, :3]


# speedup vs baseline: 8.4327x; 1.0899x over previous
import functools

import jax
import jax.numpy as jnp
from jax import lax
from jax.experimental import pallas as pl
from jax.experimental.pallas import tpu as pltpu
from jax.experimental.pallas import tpu_sc as plsc

N = 10000
E = 320000
F_IN = 128
H = 256
LATENT = 128
DENSE = 512
G = 64
EPS = 1e-5

NC = 2
NS = 16
CH = 128
IG = 8
DW = 128
EP = 327680
NROW = 10112
STRIPE = NROW // NS
R = 1000
NBLK = N // R

def _sc_mesh():
    return plsc.VectorSubcoreMesh(core_axis_name="c", subcore_axis_name="s",
                                  num_cores=NC, num_subcores=NS)


def _sc_deg(dst2d, ones_h, zero_h, dw=DW):
    chunks = EP // (NC * NS * CH)
    groups = chunks // IG

    @functools.partial(
        pl.kernel,
        out_type=jax.ShapeDtypeStruct((NC, NROW, dw), jnp.float32),
        mesh=_sc_mesh(),
        scratch_types=[
            pltpu.VMEM((IG, CH), jnp.int32),
            pltpu.VMEM((CH, dw), jnp.float32),
            pltpu.VMEM_SHARED((NROW, dw), jnp.float32),
            pltpu.SemaphoreType.DMA,
        ],
    )
    def k(dst_h, ones_hh, zero_hh, out_h, didx, ones_v, acc, sem):
        cid = lax.axis_index("c")
        sid = lax.axis_index("s")
        tid = cid * NS + sid
        pltpu.sync_copy(ones_hh, ones_v)
        pltpu.sync_copy(zero_hh, acc.at[pl.ds(sid * STRIPE, STRIPE)])
        plsc.subcore_barrier()

        def group(gi, carry):
            pltpu.sync_copy(dst_h.at[pl.ds(tid * chunks + gi * IG, IG)], didx)
            scat = [pltpu.async_copy(ones_v, acc.at[didx.at[j]], sem, add=True)
                    for j in range(IG)]
            for s in scat:
                s.wait()
            return carry

        lax.fori_loop(0, groups, group, 0)
        plsc.subcore_barrier()
        pltpu.sync_copy(acc.at[pl.ds(sid * STRIPE, STRIPE)],
                        out_h.at[cid, pl.ds(sid * STRIPE, STRIPE)])

    return k(dst2d, ones_h, zero_h)


def _make_agg(feat_split):
    chunks = EP // (NS * CH) if feat_split else EP // (NC * NS * CH)
    groups = chunks // IG

    @functools.partial(
        pl.kernel,
        out_type=jax.ShapeDtypeStruct((NC, NROW, 128), jnp.float32),
        mesh=_sc_mesh(),
        scratch_types=[
            pltpu.VMEM((IG, CH), jnp.int32),
            pltpu.VMEM((IG, CH), jnp.int32),
            pltpu.VMEM((CH, 128), jnp.float32),
            pltpu.VMEM((CH, 128), jnp.float32),
            pltpu.VMEM_SHARED((NROW, 128), jnp.float32),
            pltpu.SemaphoreType.DMA,
            pltpu.SemaphoreType.DMA,
            pltpu.SemaphoreType.DMA,
            pltpu.SemaphoreType.DMA,
        ],
    )
    def k(table_h, src_h, dst_h, zero_hh, out_h, sidx, didx, rows0, rows1,
          acc, gsem0, gsem1, ssem0, ssem1):
        cid = lax.axis_index("c")
        sid = lax.axis_index("s")
        if feat_split:
            src_base = cid * (EP // CH) + sid * chunks
            dst_base = sid * chunks
        else:
            tid = cid * NS + sid
            src_base = tid * chunks
            dst_base = tid * chunks
        pltpu.sync_copy(zero_hh, acc.at[pl.ds(sid * STRIPE, STRIPE)])
        plsc.subcore_barrier()

        rows = (rows0, rows1)
        gsem = (gsem0, gsem1)
        ssem = (ssem0, ssem1)

        def group(gi, carry):
            pltpu.sync_copy(src_h.at[pl.ds(src_base + gi * IG, IG)], sidx)
            pltpu.sync_copy(dst_h.at[pl.ds(dst_base + gi * IG, IG)], didx)
            scat = [None, None]
            for j in range(IG):
                b = j % 2
                if scat[b] is not None:
                    scat[b].wait()
                pltpu.async_copy(table_h.at[sidx.at[j]], rows[b], gsem[b]).wait()
                scat[b] = pltpu.async_copy(rows[b], acc.at[didx.at[j]],
                                           ssem[b], add=True)
            scat[0].wait()
            scat[1].wait()
            return carry

        lax.fori_loop(0, groups, group, 0)
        plsc.subcore_barrier()
        pltpu.sync_copy(acc.at[pl.ds(sid * STRIPE, STRIPE)],
                        out_h.at[cid, pl.ds(sid * STRIPE, STRIPE)])

    return k


def _tc_prep1(x, degp):
    def body(x_ref, d_ref, dinv_ref, u_ref):
        d = d_ref[...]
        dinv = lax.rsqrt(d[0, :, 0:1] + d[1, :, 0:1] + 1.0)
        dinv_ref[...] = dinv
        u_ref[...] = x_ref[...] * dinv

    return pl.pallas_call(
        body,
        grid=(NBLK,),
        in_specs=[
            pl.BlockSpec((R, F_IN), lambda i: (i, 0)),
            pl.BlockSpec((NC, R, DW), lambda i: (0, i, 0)),
        ],
        out_specs=[
            pl.BlockSpec((R, 1), lambda i: (i, 0)),
            pl.BlockSpec((R, F_IN), lambda i: (i, 0)),
        ],
        out_shape=[
            jax.ShapeDtypeStruct((N, 1), jnp.float32),
            jax.ShapeDtypeStruct((N, F_IN), jnp.float32),
        ],
    )(x, degp)


def _tc_layer_mm(parts, u, dinv, W, b, split_u, fout):
    fin = 256 if split_u else 128

    def body(p_ref, u_ref, dinv_ref, W_ref, b_ref, z_ref, st_ref, acc_ref):
        i = pl.program_id(0)
        p = p_ref[...]
        dinv = dinv_ref[...]
        if split_u:
            uu = u_ref[...]
            y = jnp.concatenate([(p[0] + uu[0]) * dinv,
                                 (p[1] + uu[1]) * dinv], axis=1)
        else:
            y = (p[0] + p[1] + u_ref[...]) * dinv
        z = jnp.dot(y, W_ref[...], preferred_element_type=jnp.float32, precision=lax.Precision.HIGHEST) + b_ref[...]
        z_ref[...] = z

        @pl.when(i == 0)
        def _():
            acc_ref[...] = jnp.zeros((8, fout), jnp.float32)

        acc_ref[0:1] += jnp.sum(z, axis=0, keepdims=True)
        acc_ref[1:2] += jnp.sum(z * z, axis=0, keepdims=True)

        @pl.when(i == NBLK - 1)
        def _():
            st_ref[...] = acc_ref[...]

    u_spec = (pl.BlockSpec((NC, R, 128), lambda i: (0, i, 0)) if split_u
              else pl.BlockSpec((R, 128), lambda i: (i, 0)))
    return pl.pallas_call(
        body,
        grid=(NBLK,),
        in_specs=[
            pl.BlockSpec((NC, R, 128), lambda i: (0, i, 0)),
            u_spec,
            pl.BlockSpec((R, 1), lambda i: (i, 0)),
            pl.BlockSpec((fin, fout), lambda i: (0, 0)),
            pl.BlockSpec((1, fout), lambda i: (0, 0)),
        ],
        out_specs=[
            pl.BlockSpec((R, fout), lambda i: (i, 0)),
            pl.BlockSpec((8, fout), lambda i: (0, 0)),
        ],
        out_shape=[
            jax.ShapeDtypeStruct((N, fout), jnp.float32),
            jax.ShapeDtypeStruct((8, fout), jnp.float32),
        ],
        scratch_shapes=[pltpu.VMEM((8, fout), jnp.float32)],
    )(parts, u, dinv, W, b)


def _tc_bn_prep2(z, st, g, be, dinv):
    def body(z_ref, st_ref, g_ref, be_ref, dinv_ref, u_ref):
        st = st_ref[...]
        mu = st[0:1] / N
        var = st[1:2] / N - mu * mu
        alpha = g_ref[...] * lax.rsqrt(var + EPS)
        beta = be_ref[...] - mu * alpha
        h = jnp.maximum(z_ref[...] * alpha + beta, 0.0)
        u = h * dinv_ref[...]
        u_ref[...] = jnp.stack([u[:, :128], u[:, 128:]], axis=0)

    return pl.pallas_call(
        body,
        grid=(NBLK,),
        in_specs=[
            pl.BlockSpec((R, H), lambda i: (i, 0)),
            pl.BlockSpec((8, H), lambda i: (0, 0)),
            pl.BlockSpec((1, H), lambda i: (0, 0)),
            pl.BlockSpec((1, H), lambda i: (0, 0)),
            pl.BlockSpec((R, 1), lambda i: (i, 0)),
        ],
        out_specs=pl.BlockSpec((NC, R, 128), lambda i: (0, i, 0)),
        out_shape=jax.ShapeDtypeStruct((NC, N, 128), jnp.float32),
    )(z, st, g, be, dinv)


def _tc_bn_mm_prep3(z, st, g, be, dinv, W3):
    def body(z_ref, st_ref, g_ref, be_ref, dinv_ref, W_ref, u_ref):
        st = st_ref[...]
        mu = st[0:1] / N
        var = st[1:2] / N - mu * mu
        alpha = g_ref[...] * lax.rsqrt(var + EPS)
        beta = be_ref[...] - mu * alpha
        h = jnp.maximum(z_ref[...] * alpha + beta, 0.0)
        t = jnp.dot(h, W_ref[...], preferred_element_type=jnp.float32, precision=lax.Precision.HIGHEST)
        u_ref[...] = t * dinv_ref[...]

    return pl.pallas_call(
        body,
        grid=(NBLK,),
        in_specs=[
            pl.BlockSpec((R, H), lambda i: (i, 0)),
            pl.BlockSpec((8, H), lambda i: (0, 0)),
            pl.BlockSpec((1, H), lambda i: (0, 0)),
            pl.BlockSpec((1, H), lambda i: (0, 0)),
            pl.BlockSpec((R, 1), lambda i: (i, 0)),
            pl.BlockSpec((H, LATENT), lambda i: (0, 0)),
        ],
        out_specs=pl.BlockSpec((R, LATENT), lambda i: (i, 0)),
        out_shape=jax.ShapeDtypeStruct((N, LATENT), jnp.float32),
    )(z, st, g, be, dinv, W3)


def _tc_layer3_pool(parts, u3, dinv, b3, batch2d):
    def body(p_ref, u_ref, dinv_ref, b_ref, bat_ref, gs_ref, cnt_ref, st_ref,
             gacc, cacc, sacc):
        i = pl.program_id(0)

        @pl.when(i == 0)
        def _():
            gacc[...] = jnp.zeros((G, LATENT), jnp.float32)
            cacc[...] = jnp.zeros((G, 1), jnp.float32)
            sacc[...] = jnp.zeros((8, LATENT), jnp.float32)

        p = p_ref[...]
        y = (p[0] + p[1] + u_ref[...]) * dinv_ref[...] + b_ref[...]
        sacc[0:1] += jnp.sum(y, axis=0, keepdims=True)
        sacc[1:2] += jnp.sum(y * y, axis=0, keepdims=True)
        bat = bat_ref[0]
        oh = (lax.broadcasted_iota(jnp.int32, (G, R), 0)
              == jnp.broadcast_to(bat, (G, R))).astype(jnp.float32)
        gacc[...] += jnp.dot(oh, y, preferred_element_type=jnp.float32, precision=lax.Precision.HIGHEST)
        cacc[...] += jnp.sum(oh, axis=1, keepdims=True)

        @pl.when(i == NBLK - 1)
        def _():
            gs_ref[...] = gacc[...]
            cnt_ref[...] = cacc[...]
            st_ref[...] = sacc[...]

    return pl.pallas_call(
        body,
        grid=(NBLK,),
        in_specs=[
            pl.BlockSpec((NC, R, 128), lambda i: (0, i, 0)),
            pl.BlockSpec((R, LATENT), lambda i: (i, 0)),
            pl.BlockSpec((R, 1), lambda i: (i, 0)),
            pl.BlockSpec((1, LATENT), lambda i: (0, 0)),
            pl.BlockSpec((1, 1, R), lambda i: (i, 0, 0)),
        ],
        out_specs=[
            pl.BlockSpec((G, LATENT), lambda i: (0, 0)),
            pl.BlockSpec((G, 1), lambda i: (0, 0)),
            pl.BlockSpec((8, LATENT), lambda i: (0, 0)),
        ],
        out_shape=[
            jax.ShapeDtypeStruct((G, LATENT), jnp.float32),
            jax.ShapeDtypeStruct((G, 1), jnp.float32),
            jax.ShapeDtypeStruct((8, LATENT), jnp.float32),
        ],
        scratch_shapes=[
            pltpu.VMEM((G, LATENT), jnp.float32),
            pltpu.VMEM((G, 1), jnp.float32),
            pltpu.VMEM((8, LATENT), jnp.float32),
        ],
    )(parts, u3, dinv, b3, batch2d)


def _tc_head(gsum, cnt, st, g3, be3, L0, Lb0, L1, Lb1, L2, Lb2, Lo_p, Lbo_p):
    def body(gs_ref, cnt_ref, st_ref, g_ref, be_ref, L0_ref, Lb0_ref, L1_ref,
             Lb1_ref, L2_ref, Lb2_ref, Lo_ref, Lbo_ref, o_ref):
        st = st_ref[...]
        mu = st[0:1] / N
        var = st[1:2] / N - mu * mu
        alpha = g_ref[...] * lax.rsqrt(var + EPS)
        beta = be_ref[...] - mu * alpha
        pooled = gs_ref[...] / jnp.maximum(cnt_ref[...], 1.0)
        hb = pooled * alpha + beta
        z = jnp.maximum(jnp.dot(hb, L0_ref[...], preferred_element_type=jnp.float32, precision=lax.Precision.HIGHEST) + Lb0_ref[...], 0.0)
        z = jnp.maximum(jnp.dot(z, L1_ref[...], preferred_element_type=jnp.float32, precision=lax.Precision.HIGHEST) + Lb1_ref[...], 0.0)
        z = jnp.maximum(jnp.dot(z, L2_ref[...], preferred_element_type=jnp.float32, precision=lax.Precision.HIGHEST) + Lb2_ref[...], 0.0)
        o_ref[...] = jnp.dot(z, Lo_ref[...], preferred_element_type=jnp.float32, precision=lax.Precision.HIGHEST) + Lbo_ref[...]

    return pl.pallas_call(
        body,
        out_shape=jax.ShapeDtypeStruct((G, 128), jnp.float32),
    )(gsum, cnt, st, g3, be3, L0, Lb0, L1, Lb1, L2, Lb2, Lo_p, Lbo_p)


def kernel(x, edge_index, batch, W1, b1, g1, be1, W2, b2, g2, be2, W3, b3,
           g3, be3, L0, Lb0, L1, Lb1, L2, Lb2, Lo, Lbo):
    src = edge_index[0]
    dst = edge_index[1]
    pad = EP - E
    src_p = jnp.concatenate([src, jnp.zeros((pad,), jnp.int32)])
    pad_dst = N + (jnp.arange(pad, dtype=jnp.int32) % (NROW - N))
    dst_p = jnp.concatenate([dst, pad_dst])
    src2d = src_p.reshape(EP // CH, CH)
    dst2d = dst_p.reshape(EP // CH, CH)
    src_off2d = jnp.concatenate([src_p, src_p + N]).reshape(2 * EP // CH, CH)

    ones_c = jnp.ones((CH, DW), jnp.float32)
    zero1 = jnp.zeros((STRIPE, DW), jnp.float32)
    zero128 = jnp.zeros((STRIPE, 128), jnp.float32)
    batch3d = batch.reshape(NBLK, 1, R)

    b1r, b2r, b3r = b1.reshape(1, H), b2.reshape(1, H), b3.reshape(1, LATENT)
    g1r, g2r, g3r = g1.reshape(1, H), g2.reshape(1, H), g3.reshape(1, LATENT)
    be1r, be2r, be3r = be1.reshape(1, H), be2.reshape(1, H), be3.reshape(1, LATENT)
    Lb0r, Lb1r, Lb2r = Lb0.reshape(1, DENSE), Lb1.reshape(1, DENSE), Lb2.reshape(1, DENSE)
    Lo_p = jnp.pad(Lo, ((0, 0), (0, 128 - Lo.shape[1])))
    Lbo_p = jnp.pad(Lbo, (0, 128 - Lbo.shape[0])).reshape(1, 128)

    agg_es = _make_agg(feat_split=False)
    agg_fs = _make_agg(feat_split=True)

    degp = _sc_deg(dst2d, ones_c, zero1)
    dinv, u1 = _tc_prep1(x, degp)

    p1 = agg_es(u1, src2d, dst2d, zero128)
    z1, st1 = _tc_layer_mm(p1, u1, dinv, W1, b1r, split_u=False, fout=H)

    u2 = _tc_bn_prep2(z1, st1, g1r, be1r, dinv)
    p2 = agg_fs(u2.reshape(2 * N, 128), src_off2d, dst2d, zero128)
    z2, st2 = _tc_layer_mm(p2, u2, dinv, W2, b2r, split_u=True, fout=H)

    u3 = _tc_bn_mm_prep3(z2, st2, g2r, be2r, dinv, W3)
    p3 = agg_es(u3, src2d, dst2d, zero128)
    gsum, cnt, st3 = _tc_layer3_pool(p3, u3, dinv, b3r, batch3d)

    out = _tc_head(gsum, cnt, st3, g3r, be3r, L0, Lb0r, L1, Lb1r, L2, Lb2r,
                   Lo_p, Lbo_p)
    return out[:, :3]
